# Initial kernel scaffold; baseline (speedup 1.0000x reference)
import jax
import jax.numpy as jnp
from jax.experimental import pallas as pl


def kernel(anchor_ids, pos_ids, neg_ids, users, items):
    raise NotImplementedError("write your pallas kernel here")


# speedup vs baseline: 3.9216x; 3.9216x over previous
import functools

import jax
import jax.numpy as jnp
from jax import lax
from jax.experimental import pallas as pl
from jax.experimental.pallas import tpu as pltpu
from jax.experimental.pallas import tpu_sc as plsc

_INFO = plsc.get_sparse_core_info()
_NC = _INFO.num_cores
_NS = _INFO.num_subcores
_L = _INFO.num_lanes
_NW = _NC * _NS

_BATCH = 4096
_NNEG = 50
_D = 128
_BPW = _BATCH // _NW
_NG = _BPW // _L


def _body(aid_hbm, pid_hbm, nidT_hbm, users_hbm, items_hbm, out_hbm,
          aidx_v, pidx_v, nidx_v, a_v, p_v, n_v, pd_v, out_v, sem):
    wid = lax.axis_index("s") * _NC + lax.axis_index("c")
    base = wid * _BPW

    pltpu.sync_copy(aid_hbm.at[pl.ds(base, _BPW)], aidx_v)
    pltpu.sync_copy(pid_hbm.at[pl.ds(base, _BPW)], pidx_v)
    pltpu.sync_copy(nidT_hbm.at[:, pl.ds(base, _BPW)], nidx_v)

    pltpu.async_copy(users_hbm.at[aidx_v], a_v, sem).wait()
    pltpu.async_copy(items_hbm.at[pidx_v], p_v, sem).wait()

    i0 = lax.iota(jnp.int32, _L)
    perms = [i0 ^ 8, i0 ^ 4, i0 ^ 2, i0 ^ 1]

    def butterfly(acc):
        for p in perms:
            acc = acc + acc.at[p].get(mode="promise_in_bounds")
        return acc

    def dist_group(y_v, g):
        res = jnp.zeros((_L,), jnp.float32)
        for rl in range(_L):
            r = g * _L + rl
            acc = jnp.zeros((_L,), jnp.float32)
            for c in range(_D // _L):
                a = a_v[r, pl.ds(c * _L, _L)]
                b = y_v[r, pl.ds(c * _L, _L)]
                d = a - b
                acc = acc + d * d
            res = jnp.where(i0 == rl, butterfly(acc), res)
        return res

    def pg(g, carry):
        pd_v[pl.ds(g * _L, _L)] = dist_group(p_v, g)
        return carry

    lax.fori_loop(0, _NG, pg, 0)

    def jbody(j, carry):
        pltpu.async_copy(items_hbm.at[nidx_v.at[j]], n_v, sem).wait()

        def ng(g, c2):
            out_v[j, pl.ds(g * _L, _L)] = (
                pd_v[pl.ds(g * _L, _L)] - dist_group(n_v, g))
            return c2

        lax.fori_loop(0, _NG, ng, 0)
        return carry

    lax.fori_loop(0, _NNEG, jbody, 0)
    pltpu.sync_copy(out_v, out_hbm.at[:, pl.ds(base, _BPW)])


@jax.jit
def _pairwise_sc(anchor_ids, pos_ids, negT_ids, users, items):
    mesh = plsc.VectorSubcoreMesh(core_axis_name="c", subcore_axis_name="s")
    fn = pl.kernel(
        _body,
        mesh=mesh,
        out_type=jax.ShapeDtypeStruct((_NNEG, _BATCH), jnp.float32),
        scratch_types=[
            pltpu.VMEM((_BPW,), jnp.int32),
            pltpu.VMEM((_BPW,), jnp.int32),
            pltpu.VMEM((_NNEG, _BPW), jnp.int32),
            pltpu.VMEM((_BPW, _D), jnp.float32),
            pltpu.VMEM((_BPW, _D), jnp.float32),
            pltpu.VMEM((_BPW, _D), jnp.float32),
            pltpu.VMEM((_BPW,), jnp.float32),
            pltpu.VMEM((_NNEG, _BPW), jnp.float32),
            pltpu.SemaphoreType.DMA,
        ],
    )
    return fn(anchor_ids, pos_ids, negT_ids, users, items)


def kernel(anchor_ids, pos_ids, neg_ids, users, items):
    negT = neg_ids.T
    outT = _pairwise_sc(anchor_ids, pos_ids, negT, users, items)
    return outT.T


# speedup vs baseline: 10.6037x; 2.7039x over previous
import jax
import jax.numpy as jnp
from jax import lax
from jax.experimental import pallas as pl
from jax.experimental.pallas import tpu as pltpu
from jax.experimental.pallas import tpu_sc as plsc

_INFO = plsc.get_sparse_core_info()
_NC = _INFO.num_cores
_NS = _INFO.num_subcores
_L = _INFO.num_lanes
_NW = _NC * _NS

_BATCH = 4096
_NNEG = 50
_D = 128
_BPW = _BATCH // _NW
_NG = _BPW // _L
_JC = 2
_NCHUNK = _NNEG // _JC


def _body(aid_hbm, pid_hbm, nidT_hbm, users_hbm, items_hbm, out_hbm,
          aidx_v, pidx_v, nidx_v, a_v, nb0, nb1, pd_v, out_v,
          sem_a, sem0, sem1):
    wid = lax.axis_index("s") * _NC + lax.axis_index("c")
    base = wid * _BPW

    pltpu.sync_copy(aid_hbm.at[pl.ds(base, _BPW)], aidx_v)
    pltpu.sync_copy(pid_hbm.at[pl.ds(base, _BPW)], pidx_v)
    pltpu.sync_copy(nidT_hbm.at[:, pl.ds(base, _BPW)], nidx_v)

    def fire_chunk(ch, buf, sem):
        for jl in range(_JC):
            pltpu.async_copy(
                items_hbm.at[nidx_v.at[ch * _JC + jl]],
                buf.at[pl.ds(jl * _BPW, _BPW)], sem)

    def drain_chunk(buf, sem):
        for jl in range(_JC):
            pltpu.make_async_copy(
                items_hbm.at[nidx_v.at[0]],
                buf.at[pl.ds(jl * _BPW, _BPW)], sem).wait()

    cp_a = pltpu.async_copy(users_hbm.at[aidx_v], a_v, sem_a)
    cp_p = pltpu.async_copy(items_hbm.at[pidx_v], nb0.at[pl.ds(0, _BPW)], sem0)
    fire_chunk(0, nb1, sem1)

    i0 = lax.iota(jnp.int32, _L)
    perms = [i0 ^ 8, i0 ^ 4, i0 ^ 2, i0 ^ 1]

    def butterfly(acc):
        for p in perms:
            acc = acc + acc.at[p].get(mode="promise_in_bounds")
        return acc

    cp_a.wait()
    cp_p.wait()

    def pg(g, carry):
        res = jnp.zeros((_L,), jnp.float32)
        for rl in range(_L):
            r = g * _L + rl
            acc = jnp.zeros((_L,), jnp.float32)
            for c in range(_D // _L):
                a = a_v[r, pl.ds(c * _L, _L)]
                b = nb0[r, pl.ds(c * _L, _L)]
                d = a - b
                acc = acc + d * d
            res = jnp.where(i0 == rl, butterfly(acc), res)
        pd_v[pl.ds(g * _L, _L)] = res
        return carry

    lax.fori_loop(0, _NG, pg, 0)

    def chunk_compute(buf, ch):
        def ng(g, c2):
            res = [jnp.zeros((_L,), jnp.float32) for _ in range(_JC)]
            for rl in range(_L):
                r = g * _L + rl
                a = [a_v[r, pl.ds(c * _L, _L)] for c in range(_D // _L)]
                for jl in range(_JC):
                    acc = jnp.zeros((_L,), jnp.float32)
                    for c in range(_D // _L):
                        b = buf[jl * _BPW + r, pl.ds(c * _L, _L)]
                        d = a[c] - b
                        acc = acc + d * d
                    res[jl] = jnp.where(i0 == rl, butterfly(acc), res[jl])
            pd = pd_v[pl.ds(g * _L, _L)]
            for jl in range(_JC):
                out_v[ch * _JC + jl, pl.ds(g * _L, _L)] = pd - res[jl]
            return c2

        lax.fori_loop(0, _NG, ng, 0)

    def pair(m, carry):
        ch0 = m * 2

        @pl.when(ch0 + 1 < _NCHUNK)
        def _():
            fire_chunk(ch0 + 1, nb0, sem0)

        drain_chunk(nb1, sem1)
        chunk_compute(nb1, ch0)

        @pl.when(ch0 + 2 < _NCHUNK)
        def _():
            fire_chunk(ch0 + 2, nb1, sem1)

        @pl.when(ch0 + 1 < _NCHUNK)
        def _():
            drain_chunk(nb0, sem0)
            chunk_compute(nb0, ch0 + 1)

        return carry

    lax.fori_loop(0, (_NCHUNK + 1) // 2, pair, 0)

    pltpu.sync_copy(out_v, out_hbm.at[:, pl.ds(base, _BPW)])


@jax.jit
def _pairwise_sc(anchor_ids, pos_ids, negT_ids, users, items):
    mesh = plsc.VectorSubcoreMesh(core_axis_name="c", subcore_axis_name="s")
    fn = pl.kernel(
        _body,
        mesh=mesh,
        out_type=jax.ShapeDtypeStruct((_NNEG, _BATCH), jnp.float32),
        scratch_types=[
            pltpu.VMEM((_BPW,), jnp.int32),
            pltpu.VMEM((_BPW,), jnp.int32),
            pltpu.VMEM((_NNEG, _BPW), jnp.int32),
            pltpu.VMEM((_BPW, _D), jnp.float32),
            pltpu.VMEM((_JC * _BPW, _D), jnp.float32),
            pltpu.VMEM((_JC * _BPW, _D), jnp.float32),
            pltpu.VMEM((_BPW,), jnp.float32),
            pltpu.VMEM((_NNEG, _BPW), jnp.float32),
            pltpu.SemaphoreType.DMA,
            pltpu.SemaphoreType.DMA,
            pltpu.SemaphoreType.DMA,
        ],
    )
    return fn(anchor_ids, pos_ids, negT_ids, users, items)


def kernel(anchor_ids, pos_ids, neg_ids, users, items):
    negT = neg_ids.T
    outT = _pairwise_sc(anchor_ids, pos_ids, negT, users, items)
    return outT.T
